# Initial kernel scaffold; baseline (speedup 1.0000x reference)
import functools

import jax
import jax.numpy as jnp
from jax import lax
from jax.experimental import pallas as pl
from jax.experimental.pallas import tpu as pltpu
from jax.experimental.pallas import tpu_sc as plsc

N_NODES = 1_000_000
DIM = 32
B0, B1 = 4096, 200
TOTAL = B0 * B1

NC, NS = 2, 16
NW = NC * NS
PER_W = TOTAL // NW

CH = 128
NCH = PER_W // CH
G = 10
NG = NCH // G
GROUP_ROWS = G * CH


def _sc_gather(idx2, table):
    mesh = plsc.VectorSubcoreMesh(
        core_axis_name="c", subcore_axis_name="s",
        num_cores=NC, num_subcores=NS)

    @functools.partial(
        pl.kernel,
        out_type=jax.ShapeDtypeStruct((TOTAL, DIM), jnp.float32),
        mesh=mesh,
        scratch_types=[
            pltpu.VMEM((NCH, CH), jnp.int32),
            pltpu.VMEM((GROUP_ROWS, DIM), jnp.float32),
            pltpu.SemaphoreType.DMA,
        ],
    )
    def k(idx_hbm, table_hbm, out_hbm, idx_v, rows_v, gsem):
        wid = lax.axis_index("s") * NC + lax.axis_index("c")
        row_base = wid * PER_W
        pltpu.sync_copy(idx_hbm.at[pl.ds(wid * NCH, NCH)], idx_v)

        def group(g, carry):
            descs = []
            for b in range(G):
                c = g * G + b
                descs.append(pltpu.async_copy(
                    table_hbm.at[idx_v.at[c]],
                    rows_v.at[pl.ds(b * CH, CH)],
                    gsem))
            for d in descs:
                d.wait()
            pltpu.sync_copy(
                rows_v,
                out_hbm.at[pl.ds(row_base + g * GROUP_ROWS, GROUP_ROWS)])
            return carry

        lax.fori_loop(0, NG, group, 0)

    return k(idx2, table)


def kernel(indices, embeddings):
    idx2 = indices.reshape(NW * NCH, CH)
    out = _sc_gather(idx2, embeddings)
    return out.reshape(B0, B1, DIM)


# speedup vs baseline: 1.4835x; 1.4835x over previous
import functools

import jax
import jax.numpy as jnp
from jax import lax
from jax.experimental import pallas as pl
from jax.experimental.pallas import tpu as pltpu
from jax.experimental.pallas import tpu_sc as plsc

N_NODES = 1_000_000
DIM = 32
B0, B1 = 4096, 200
TOTAL = B0 * B1

NC, NS = 2, 16
NW = NC * NS
PER_W = TOTAL // NW

CH = 128
NCH = PER_W // CH
G = 10
NG = NCH // G
GROUP_ROWS = G * CH


def _sc_gather(idx2, table):
    mesh = plsc.VectorSubcoreMesh(
        core_axis_name="c", subcore_axis_name="s",
        num_cores=NC, num_subcores=NS)

    @functools.partial(
        pl.kernel,
        out_type=jax.ShapeDtypeStruct((TOTAL, DIM), jnp.float32),
        mesh=mesh,
        scratch_types=[
            pltpu.VMEM((NCH, CH), jnp.int32),
            pltpu.VMEM((GROUP_ROWS, DIM), jnp.float32),
            pltpu.SemaphoreType.DMA,
        ],
        compiler_params=pltpu.CompilerParams(use_tc_tiling_on_sc=False),
    )
    def k(idx_hbm, table_hbm, out_hbm, idx_v, rows_v, gsem):
        wid = lax.axis_index("s") * NC + lax.axis_index("c")
        row_base = wid * PER_W
        pltpu.sync_copy(idx_hbm.at[pl.ds(wid * NCH, NCH)], idx_v)

        def group(g, carry):
            descs = []
            for b in range(G):
                c = g * G + b
                descs.append(pltpu.async_copy(
                    table_hbm.at[idx_v.at[c]],
                    rows_v.at[pl.ds(b * CH, CH)],
                    gsem))
            for d in descs:
                d.wait()
            pltpu.sync_copy(
                rows_v,
                out_hbm.at[pl.ds(row_base + g * GROUP_ROWS, GROUP_ROWS)])
            return carry

        lax.fori_loop(0, NG, group, 0)

    return k(idx2, table)


def kernel(indices, embeddings):
    idx2 = indices.reshape(NW * NCH, CH)
    out = _sc_gather(idx2, embeddings)
    return out.reshape(B0, B1, DIM)


# speedup vs baseline: 1.5021x; 1.0126x over previous
import functools

import jax
import jax.numpy as jnp
from jax import lax
from jax.experimental import pallas as pl
from jax.experimental.pallas import tpu as pltpu
from jax.experimental.pallas import tpu_sc as plsc

N_NODES = 1_000_000
DIM = 32
B0, B1 = 4096, 200
TOTAL = B0 * B1

NC, NS = 2, 16
NW = NC * NS
PER_W = TOTAL // NW

CH = 128
NCH = PER_W // CH
G = 10
NG = NCH // G
GROUP_ROWS = G * CH
NG2 = NG // 2


def _sc_gather(idx2, table):
    mesh = plsc.VectorSubcoreMesh(
        core_axis_name="c", subcore_axis_name="s",
        num_cores=NC, num_subcores=NS)

    @functools.partial(
        pl.kernel,
        out_type=jax.ShapeDtypeStruct((TOTAL, DIM), jnp.float32),
        mesh=mesh,
        scratch_types=[
            pltpu.VMEM((NCH, CH), jnp.int32),
            pltpu.VMEM((2, GROUP_ROWS, DIM), jnp.float32),
            pltpu.SemaphoreType.DMA,
            pltpu.SemaphoreType.DMA,
            pltpu.SemaphoreType.DMA,
            pltpu.SemaphoreType.DMA,
        ],
        compiler_params=pltpu.CompilerParams(use_tc_tiling_on_sc=False),
    )
    def k(idx_hbm, table_hbm, out_hbm, idx_v, rows_v, gsem0, gsem1,
          ssem0, ssem1):
        gsems = (gsem0, gsem1)
        ssems = (ssem0, ssem1)
        wid = lax.axis_index("s") * NC + lax.axis_index("c")
        row_base = wid * PER_W
        pltpu.sync_copy(idx_hbm.at[pl.ds(wid * NCH, NCH)], idx_v)

        def fire(g, buf):
            for b in range(G):
                pltpu.async_copy(
                    table_hbm.at[idx_v.at[g * G + b]],
                    rows_v.at[buf, pl.ds(b * CH, CH)],
                    gsems[buf])

        def drain(buf):
            pltpu.make_async_copy(
                out_hbm.at[pl.ds(0, GROUP_ROWS)],
                rows_v.at[buf],
                gsems[buf]).wait()

        def start_store(g, buf):
            pltpu.async_copy(
                rows_v.at[buf],
                out_hbm.at[pl.ds(row_base + g * GROUP_ROWS, GROUP_ROWS)],
                ssems[buf])

        def wait_store(buf):
            pltpu.make_async_copy(
                rows_v.at[buf],
                out_hbm.at[pl.ds(0, GROUP_ROWS)],
                ssems[buf]).wait()

        fire(0, 0)
        fire(1, 1)
        drain(0)
        start_store(0, 0)

        def body(s2, carry):
            g0 = 2 * s2
            g1 = g0 + 1
            wait_store(0)
            fire(g0, 0)
            drain(1)
            start_store(g0 - 1, 1)
            wait_store(1)
            fire(g1, 1)
            drain(0)
            start_store(g0, 0)
            return carry

        lax.fori_loop(1, NG2, body, 0)

        drain(1)
        start_store(NG - 1, 1)
        wait_store(0)
        wait_store(1)

    return k(idx2, table)


def kernel(indices, embeddings):
    idx2 = indices.reshape(NW * NCH, CH)
    out = _sc_gather(idx2, embeddings)
    return out.reshape(B0, B1, DIM)
